# baseline (speedup 1.0000x reference)
import functools

import jax
import jax.numpy as jnp
from jax import lax
from jax.experimental import pallas as pl
from jax.experimental.pallas import tpu as pltpu
from jax.experimental.pallas import tpu_sc as plsc

L = 200
LP = 256
NC = 2
NS = 16
NW = NC * NS
RPW = LP // NW
SKILL = 128
KH = 64
D = SKILL * KH
NG = SKILL // 16


def _sc_body(sq_hbm, item_hbm, u_hbm, out_hbm, idx_v, rows_v, u_v, out_v, sem):
    wid = lax.axis_index("s") * NC + lax.axis_index("c")
    base = wid * RPW
    pltpu.sync_copy(sq_hbm.at[pl.ds(base, RPW)], idx_v)
    cp = pltpu.async_copy(item_hbm.at[idx_v], rows_v, sem)
    pltpu.sync_copy(u_hbm, u_v)
    cp.wait()
    lanes = lax.iota(jnp.int32, 16)
    for g in range(NG):
        sidx = (g * 16 + lanes) * KH

        def body(kh, accs, sidx=sidx):
            col = sidx + kh
            uv = plsc.load_gather(u_v, [col])
            return tuple(
                accs[r] + uv * plsc.load_gather(
                    rows_v, [jnp.full((16,), r, jnp.int32), col])
                for r in range(RPW)
            )

        accs = lax.fori_loop(
            0, KH, body,
            tuple(jnp.zeros((16,), jnp.float32) for _ in range(RPW)))
        for r in range(RPW):
            out_v[r, pl.ds(g * 16, 16)] = accs[r]
    pltpu.sync_copy(out_v, out_hbm.at[pl.ds(base, RPW)])


_sc_gather_dot = functools.partial(
    pl.kernel,
    out_type=jax.ShapeDtypeStruct((LP, SKILL), jnp.float32),
    mesh=plsc.VectorSubcoreMesh(
        core_axis_name="c", subcore_axis_name="s",
        num_cores=NC, num_subcores=NS),
    scratch_types=[
        pltpu.VMEM((LP // NW,), jnp.int32),
        pltpu.VMEM((RPW, D), jnp.float32),
        pltpu.VMEM((D,), jnp.float32),
        pltpu.VMEM((RPW, SKILL), jnp.float32),
        pltpu.SemaphoreType.DMA,
    ],
    compiler_params=pltpu.CompilerParams(needs_layout_passes=False),
)(_sc_body)


def _tc_body(imp_ref, tk_ref, out_ref):
    imp = jnp.maximum(imp_ref[...], 0.0)
    i = lax.broadcasted_iota(jnp.int32, (LP, LP), 0)
    j = lax.broadcasted_iota(jnp.int32, (LP, LP), 1)
    a = jnp.where((j < i) & (j < L), 1.0, 0.0)
    acc = jnp.dot(a, imp, preferred_element_type=jnp.float32)
    out_ref[...] = jax.nn.sigmoid(acc + tk_ref[...])


def _tc_epilogue(improves, tk):
    return pl.pallas_call(
        _tc_body,
        out_shape=jax.ShapeDtypeStruct((LP, SKILL), jnp.float32),
    )(improves, tk)


def kernel(user, sq, user_initial_k, user_improving_k, item_improving_k):
    sq32 = sq.astype(jnp.int32)
    sq_pad = jnp.concatenate(
        [sq32, jnp.zeros((LP - L,), jnp.int32)])
    item_flat = item_improving_k.reshape(item_improving_k.shape[0], D)
    u_flat = user_improving_k[user].reshape(D)
    tk = user_initial_k[user].reshape(1, SKILL)
    improves = _sc_gather_dot(sq_pad, item_flat, u_flat)
    out_full = _tc_epilogue(improves, tk)
    return (out_full[: L + 1], 0, 0)


# speedup vs baseline: 9.3694x; 9.3694x over previous
import functools

import jax
import jax.numpy as jnp
from jax import lax
from jax.experimental import pallas as pl
from jax.experimental.pallas import tpu as pltpu
from jax.experimental.pallas import tpu_sc as plsc

L = 200
LP = 256
NC = 2
NS = 16
NW = NC * NS
RPW = LP // NW
SKILL = 128
KH = 64
NLANE = 16


def _sc_body(sq_hbm, item_hbm, u_hbm, out_hbm, idx_v, rows_v, u_v, out_v, sem):
    wid = lax.axis_index("s") * NC + lax.axis_index("c")
    base = wid * RPW
    pltpu.sync_copy(sq_hbm.at[pl.ds(base, RPW)], idx_v)
    cp = pltpu.async_copy(item_hbm.at[idx_v], rows_v, sem)
    pltpu.sync_copy(u_hbm, u_v)
    cp.wait()

    for sg in range(SKILL // NLANE):
        c0 = sg * NLANE

        def body(kh, accs, c0=c0):
            uv = u_v[kh, pl.ds(c0, NLANE)]
            return tuple(
                accs[r] + uv * rows_v[r, kh, pl.ds(c0, NLANE)]
                for r in range(RPW)
            )

        accs = lax.fori_loop(
            0, KH, body,
            tuple(jnp.zeros((NLANE,), jnp.float32) for _ in range(RPW)))
        for r in range(RPW):
            out_v[r, pl.ds(c0, NLANE)] = accs[r]

    pltpu.sync_copy(out_v, out_hbm.at[pl.ds(base, RPW)])


_sc_gather_dot = functools.partial(
    pl.kernel,
    out_type=jax.ShapeDtypeStruct((LP, SKILL), jnp.float32),
    mesh=plsc.VectorSubcoreMesh(
        core_axis_name="c", subcore_axis_name="s",
        num_cores=NC, num_subcores=NS),
    scratch_types=[
        pltpu.VMEM((RPW,), jnp.int32),
        pltpu.VMEM((RPW, KH, SKILL), jnp.float32),
        pltpu.VMEM((KH, SKILL), jnp.float32),
        pltpu.VMEM((RPW, SKILL), jnp.float32),
        pltpu.SemaphoreType.DMA,
    ],
    compiler_params=pltpu.CompilerParams(needs_layout_passes=False),
)(_sc_body)


def _tc_body(imp_ref, tk_ref, out_ref):
    imp = jnp.maximum(imp_ref[...], 0.0)
    i = lax.broadcasted_iota(jnp.int32, (LP, LP), 0)
    j = lax.broadcasted_iota(jnp.int32, (LP, LP), 1)
    a = jnp.where((j < i) & (j < L), 1.0, 0.0)
    acc = jnp.dot(a, imp, preferred_element_type=jnp.float32)
    out_ref[...] = jax.nn.sigmoid(acc + tk_ref[...])


def _tc_epilogue(improves, tk):
    return pl.pallas_call(
        _tc_body,
        out_shape=jax.ShapeDtypeStruct((LP, SKILL), jnp.float32),
    )(improves, tk)


def kernel(user, sq, user_initial_k, user_improving_k, item_improving_k):
    sq32 = sq.astype(jnp.int32)
    sq_pad = jnp.concatenate([sq32, jnp.zeros((LP - L,), jnp.int32)])
    itemT = jnp.swapaxes(item_improving_k, 1, 2)
    uT = user_improving_k[user].T
    tk = user_initial_k[user][None]
    improves = _sc_gather_dot(sq_pad, itemT, uT)
    out_full = _tc_epilogue(improves, tk)
    return (out_full[: L + 1], 0, 0)


# speedup vs baseline: 10.5279x; 1.1236x over previous
import functools

import jax
import jax.numpy as jnp
from jax import lax
from jax.experimental import pallas as pl
from jax.experimental.pallas import tpu as pltpu
from jax.experimental.pallas import tpu_sc as plsc

L = 200
LP = 256
NC = 2
NS = 16
NW = NC * NS
RPW = LP // NW
NACT = L // RPW
SKILL = 128
KH = 64
NLANE = 16


def _sc_body(sq_hbm, item_hbm, uimp_hbm, user_hbm, out_hbm,
             idx_v, uidx_v, rows_v, u_v, out_v, sem, usem):
    wid = lax.axis_index("s") * NC + lax.axis_index("c")
    base = wid * RPW

    @pl.when(wid < NACT)
    def _():
        pltpu.sync_copy(sq_hbm.at[pl.ds(base, RPW)], idx_v)
        cp = pltpu.async_copy(item_hbm.at[idx_v], rows_v, sem)
        pltpu.sync_copy(user_hbm, uidx_v)
        pltpu.async_copy(uimp_hbm.at[uidx_v], u_v, usem).wait()
        cp.wait()

        def sg_body(sg, carry):
            c0 = sg * NLANE

            def kh_body(i, accs):
                kh = i * 2
                uv0 = u_v[0, kh, pl.ds(c0, NLANE)]
                uv1 = u_v[0, kh + 1, pl.ds(c0, NLANE)]
                out = []
                for r in range(RPW):
                    a = accs[r] + uv0 * rows_v[r, kh, pl.ds(c0, NLANE)]
                    out.append(a + uv1 * rows_v[r, kh + 1, pl.ds(c0, NLANE)])
                return tuple(out)

            accs = lax.fori_loop(
                0, KH // 2, kh_body,
                tuple(jnp.zeros((NLANE,), jnp.float32) for _ in range(RPW)))
            for r in range(RPW):
                out_v[r, pl.ds(c0, NLANE)] = accs[r]
            return carry

        lax.fori_loop(0, SKILL // NLANE, sg_body, 0)
        pltpu.sync_copy(out_v, out_hbm.at[pl.ds(base, RPW)])


_sc_gather_dot = functools.partial(
    pl.kernel,
    out_type=jax.ShapeDtypeStruct((LP, SKILL), jnp.float32),
    mesh=plsc.VectorSubcoreMesh(
        core_axis_name="c", subcore_axis_name="s",
        num_cores=NC, num_subcores=NS),
    scratch_types=[
        pltpu.VMEM((RPW,), jnp.int32),
        pltpu.VMEM((1,), jnp.int32),
        pltpu.VMEM((RPW, KH, SKILL), jnp.float32),
        pltpu.VMEM((1, KH, SKILL), jnp.float32),
        pltpu.VMEM((RPW, SKILL), jnp.float32),
        pltpu.SemaphoreType.DMA,
        pltpu.SemaphoreType.DMA,
    ],
    compiler_params=pltpu.CompilerParams(needs_layout_passes=False),
)(_sc_body)


def _tc_body(imp_ref, tk_ref, out_ref):
    i = lax.broadcasted_iota(jnp.int32, (LP, LP), 0)
    j = lax.broadcasted_iota(jnp.int32, (LP, LP), 1)
    a = jnp.where(j < i, 1.0, 0.0)
    ri = lax.broadcasted_iota(jnp.int32, (LP, SKILL), 0)
    imp = jnp.where(ri < L, jnp.maximum(imp_ref[...], 0.0), 0.0)
    acc = jnp.dot(a, imp, preferred_element_type=jnp.float32)
    out_ref[...] = jax.nn.sigmoid(acc + tk_ref[...])


def _tc_epilogue(improves, tk):
    return pl.pallas_call(
        _tc_body,
        out_shape=jax.ShapeDtypeStruct((LP, SKILL), jnp.float32),
    )(improves, tk)


def kernel(user, sq, user_initial_k, user_improving_k, item_improving_k):
    sq32 = sq.astype(jnp.int32)
    itemT = jnp.swapaxes(item_improving_k, 1, 2)
    uimpT = jnp.swapaxes(user_improving_k, 1, 2)
    user_arr = jnp.asarray(user, jnp.int32)[None]
    tk = user_initial_k[user][None]
    improves = _sc_gather_dot(sq32, itemT, uimpT, user_arr)
    out_full = _tc_epilogue(improves, tk)
    return (out_full[: L + 1], 0, 0)


# speedup vs baseline: 10.7329x; 1.0195x over previous
import functools

import jax
import jax.numpy as jnp
from jax import lax
from jax.experimental import pallas as pl
from jax.experimental.pallas import tpu as pltpu
from jax.experimental.pallas import tpu_sc as plsc

L = 200
LP = 256
NC = 2
NS = 16
NW = NC * NS
RPW = LP // NW
HRPW = RPW // 2
NACT = L // RPW
SKILL = 128
KH = 64
NLANE = 16
UNROLL = 4


def _sc_body(sq_hbm, item_hbm, uimp_hbm, user_hbm, out_hbm,
             idx_v, uidx_v, rows_v, u_v, out_v, sem0, sem1, usem):
    wid = lax.axis_index("s") * NC + lax.axis_index("c")
    base = wid * RPW

    @pl.when(wid < NACT)
    def _():
        pltpu.sync_copy(sq_hbm.at[pl.ds(base, RPW)], idx_v)
        cp0 = pltpu.async_copy(item_hbm.at[idx_v], rows_v, sem0)
        pltpu.sync_copy(user_hbm, uidx_v)
        pltpu.async_copy(uimp_hbm.at[uidx_v], u_v, usem).wait()

        def half(r0):
            def sg_body(sg, carry):
                c0 = sg * NLANE

                def kh_body(i, accs):
                    out = list(accs)
                    for k in range(UNROLL):
                        kh = i * UNROLL + k
                        uv = u_v[0, kh, pl.ds(c0, NLANE)]
                        for r in range(HRPW):
                            out[r] = out[r] + uv * rows_v[
                                r0 + r, kh, pl.ds(c0, NLANE)]
                    return tuple(out)

                accs = lax.fori_loop(
                    0, KH // UNROLL, kh_body,
                    tuple(jnp.zeros((NLANE,), jnp.float32)
                          for _ in range(HRPW)))
                for r in range(HRPW):
                    out_v[r0 + r, pl.ds(c0, NLANE)] = accs[r]
                return carry

            lax.fori_loop(0, SKILL // NLANE, sg_body, 0)

        cp0.wait()
        half(0)
        half(HRPW)
        pltpu.sync_copy(out_v, out_hbm.at[pl.ds(base, RPW)])


_sc_gather_dot = functools.partial(
    pl.kernel,
    out_type=jax.ShapeDtypeStruct((LP, SKILL), jnp.float32),
    mesh=plsc.VectorSubcoreMesh(
        core_axis_name="c", subcore_axis_name="s",
        num_cores=NC, num_subcores=NS),
    scratch_types=[
        pltpu.VMEM((RPW,), jnp.int32),
        pltpu.VMEM((1,), jnp.int32),
        pltpu.VMEM((RPW, KH, SKILL), jnp.float32),
        pltpu.VMEM((1, KH, SKILL), jnp.float32),
        pltpu.VMEM((RPW, SKILL), jnp.float32),
        pltpu.SemaphoreType.DMA,
        pltpu.SemaphoreType.DMA,
        pltpu.SemaphoreType.DMA,
    ],
    compiler_params=pltpu.CompilerParams(needs_layout_passes=False),
)(_sc_body)


def _tc_body(imp_ref, tk_ref, out_ref):
    i = lax.broadcasted_iota(jnp.int32, (LP, LP), 0)
    j = lax.broadcasted_iota(jnp.int32, (LP, LP), 1)
    a = jnp.where(j < i, 1.0, 0.0)
    ri = lax.broadcasted_iota(jnp.int32, (LP, SKILL), 0)
    imp = jnp.where(ri < L, jnp.maximum(imp_ref[...], 0.0), 0.0)
    acc = jnp.dot(a, imp, preferred_element_type=jnp.float32)
    out_ref[...] = jax.nn.sigmoid(acc + tk_ref[...])[: L + 1]


def _tc_epilogue(improves, tk):
    return pl.pallas_call(
        _tc_body,
        out_shape=jax.ShapeDtypeStruct((L + 1, SKILL), jnp.float32),
    )(improves, tk)


def kernel(user, sq, user_initial_k, user_improving_k, item_improving_k):
    sq32 = sq.astype(jnp.int32)
    itemT = jnp.swapaxes(item_improving_k, 1, 2)
    uimpT = jnp.swapaxes(user_improving_k, 1, 2)
    user_arr = jnp.asarray(user, jnp.int32)[None]
    tk = user_initial_k[user][None]
    improves = _sc_gather_dot(sq32, itemT, uimpT, user_arr)
    out = _tc_epilogue(improves, tk)
    return (out, 0, 0)
